# baseline (device time: 15717 ns/iter reference)
import jax
import jax.numpy as jnp
from jax import lax
from jax.experimental import pallas as pl
from jax.experimental.pallas import tpu as pltpu

B = 512
HALF = 256
K = 8
CH = HALF // K


def kernel(x):
    x = pltpu.with_memory_space_constraint(x, pltpu.MemorySpace.HBM)

    def body(x_ref, out_ref, xstage, comm, x_ready, diag_sem,
             in_sem, copy_a, copy_b, send_a, recv_a, send_b, recv_b):
        my_x = lax.axis_index("x")
        my_y = lax.axis_index("y")
        peer_x = 1 - my_x
        peer_y = 1 - my_y

        barrier_sem = pltpu.get_barrier_semaphore()
        pl.semaphore_signal(
            barrier_sem,
            inc=1,
            device_id=(my_x, peer_y),
            device_id_type=pl.DeviceIdType.MESH,
        )
        pl.semaphore_signal(
            x_ready,
            inc=1,
            device_id=(peer_x, my_y),
            device_id_type=pl.DeviceIdType.MESH,
        )

        stage_in = pltpu.make_async_copy(x_ref, xstage, in_sem)
        stage_in.start()

        pl.semaphore_wait(barrier_sem, 1)
        stage_in.wait()

        diag = pltpu.make_async_copy(
            xstage.at[:, pl.ds(my_y * B, B)],
            out_ref.at[pl.ds(my_y * B, B), :],
            diag_sem,
        )
        diag.start()

        rdma_a = []
        for c in range(K):
            a = pltpu.make_async_remote_copy(
                src_ref=xstage.at[pl.ds(my_x * HALF + c * CH, CH),
                                  pl.ds(peer_y * B, B)],
                dst_ref=comm.at[pl.ds(my_x * HALF + c * CH, CH), :],
                send_sem=send_a.at[c],
                recv_sem=recv_a.at[c],
                device_id=(my_x, peer_y),
                device_id_type=pl.DeviceIdType.MESH,
            )
            a.start()
            rdma_a.append(a)

        rdma_b = []
        out_cp = []
        for c in range(K):
            rdma_a[c].wait_recv()
            if c == 0:
                pl.semaphore_wait(x_ready, 1)
            r = my_x * HALF + c * CH
            b = pltpu.make_async_remote_copy(
                src_ref=comm.at[pl.ds(r, CH), :],
                dst_ref=comm.at[pl.ds(r, CH), :],
                send_sem=send_b.at[c],
                recv_sem=recv_b.at[c],
                device_id=(peer_x, my_y),
                device_id_type=pl.DeviceIdType.MESH,
            )
            b.start()
            rdma_b.append(b)
            cp = pltpu.make_async_copy(
                comm.at[pl.ds(r, CH), :],
                out_ref.at[pl.ds(peer_y * B + r, CH), :],
                copy_a.at[c],
            )
            cp.start()
            out_cp.append(cp)

        for c in range(K):
            rdma_b[c].wait_recv()
            r = peer_x * HALF + c * CH
            cp = pltpu.make_async_copy(
                comm.at[pl.ds(r, CH), :],
                out_ref.at[pl.ds(peer_y * B + r, CH), :],
                copy_b.at[c],
            )
            cp.start()
            out_cp.append(cp)

        diag.wait()
        for cp in out_cp:
            cp.wait()
        for c in range(K):
            rdma_a[c].wait_send()
            rdma_b[c].wait_send()

    out = pl.pallas_call(
        body,
        out_shape=jax.ShapeDtypeStruct((2 * B, B), jnp.float32),
        in_specs=[pl.BlockSpec(memory_space=pl.ANY)],
        out_specs=pl.BlockSpec(memory_space=pl.ANY),
        scratch_shapes=[
            pltpu.VMEM((B, 2 * B), jnp.float32),
            pltpu.VMEM((B, B), jnp.float32),
            pltpu.SemaphoreType.REGULAR,
            pltpu.SemaphoreType.DMA,
            pltpu.SemaphoreType.DMA,
            pltpu.SemaphoreType.DMA((K,)),
            pltpu.SemaphoreType.DMA((K,)),
            pltpu.SemaphoreType.DMA((K,)),
            pltpu.SemaphoreType.DMA((K,)),
            pltpu.SemaphoreType.DMA((K,)),
            pltpu.SemaphoreType.DMA((K,)),
        ],
        compiler_params=pltpu.CompilerParams(collective_id=0),
    )(x)
    return out


# device time: 15018 ns/iter; 1.0465x vs baseline; 1.0465x over previous
import jax
import jax.numpy as jnp
from jax import lax
from jax.experimental import pallas as pl
from jax.experimental.pallas import tpu as pltpu

B = 512
HALF = 256
K = 8
CH = HALF // K


def kernel(x):
    x = pltpu.with_memory_space_constraint(x, pltpu.MemorySpace.HBM)

    def body(x_ref, out_ref, sendbuf, comm, x_ready, diag_sem,
             in_sem, copy_a, copy_b, send_a, recv_a, send_b, recv_b):
        my_x = lax.axis_index("x")
        my_y = lax.axis_index("y")
        peer_x = 1 - my_x
        peer_y = 1 - my_y

        barrier_sem = pltpu.get_barrier_semaphore()
        pl.semaphore_signal(
            barrier_sem,
            inc=1,
            device_id=(my_x, peer_y),
            device_id_type=pl.DeviceIdType.MESH,
        )
        pl.semaphore_signal(
            x_ready,
            inc=1,
            device_id=(peer_x, my_y),
            device_id_type=pl.DeviceIdType.MESH,
        )

        stage_in = pltpu.make_async_copy(
            x_ref.at[pl.ds(my_x * HALF, HALF), pl.ds(peer_y * B, B)],
            sendbuf,
            in_sem,
        )
        stage_in.start()

        diag = pltpu.make_async_copy(
            x_ref.at[:, pl.ds(my_y * B, B)],
            out_ref.at[pl.ds(my_y * B, B), :],
            diag_sem,
        )
        diag.start()

        pl.semaphore_wait(barrier_sem, 1)
        stage_in.wait()

        rdma_a = []
        for c in range(K):
            a = pltpu.make_async_remote_copy(
                src_ref=sendbuf.at[pl.ds(c * CH, CH), :],
                dst_ref=comm.at[pl.ds(my_x * HALF + c * CH, CH), :],
                send_sem=send_a.at[c],
                recv_sem=recv_a.at[c],
                device_id=(my_x, peer_y),
                device_id_type=pl.DeviceIdType.MESH,
            )
            a.start()
            rdma_a.append(a)

        rdma_b = []
        out_cp = []
        for c in range(K):
            rdma_a[c].wait_recv()
            if c == 0:
                pl.semaphore_wait(x_ready, 1)
            r = my_x * HALF + c * CH
            b = pltpu.make_async_remote_copy(
                src_ref=comm.at[pl.ds(r, CH), :],
                dst_ref=comm.at[pl.ds(r, CH), :],
                send_sem=send_b.at[c],
                recv_sem=recv_b.at[c],
                device_id=(peer_x, my_y),
                device_id_type=pl.DeviceIdType.MESH,
            )
            b.start()
            rdma_b.append(b)
            cp = pltpu.make_async_copy(
                comm.at[pl.ds(r, CH), :],
                out_ref.at[pl.ds(peer_y * B + r, CH), :],
                copy_a.at[c],
            )
            cp.start()
            out_cp.append(cp)

        for c in range(K):
            rdma_b[c].wait_recv()
            r = peer_x * HALF + c * CH
            cp = pltpu.make_async_copy(
                comm.at[pl.ds(r, CH), :],
                out_ref.at[pl.ds(peer_y * B + r, CH), :],
                copy_b.at[c],
            )
            cp.start()
            out_cp.append(cp)

        diag.wait()
        for cp in out_cp:
            cp.wait()
        for c in range(K):
            rdma_a[c].wait_send()
            rdma_b[c].wait_send()

    out = pl.pallas_call(
        body,
        out_shape=jax.ShapeDtypeStruct((2 * B, B), jnp.float32),
        in_specs=[pl.BlockSpec(memory_space=pl.ANY)],
        out_specs=pl.BlockSpec(memory_space=pl.ANY),
        scratch_shapes=[
            pltpu.VMEM((HALF, B), jnp.float32),
            pltpu.VMEM((B, B), jnp.float32),
            pltpu.SemaphoreType.REGULAR,
            pltpu.SemaphoreType.DMA,
            pltpu.SemaphoreType.DMA,
            pltpu.SemaphoreType.DMA((K,)),
            pltpu.SemaphoreType.DMA((K,)),
            pltpu.SemaphoreType.DMA((K,)),
            pltpu.SemaphoreType.DMA((K,)),
            pltpu.SemaphoreType.DMA((K,)),
            pltpu.SemaphoreType.DMA((K,)),
        ],
        compiler_params=pltpu.CompilerParams(collective_id=0),
    )(x)
    return out


# device time: 14844 ns/iter; 1.0588x vs baseline; 1.0117x over previous
import jax
import jax.numpy as jnp
from jax import lax
from jax.experimental import pallas as pl
from jax.experimental.pallas import tpu as pltpu

B = 512
HALF = 256
K = 8
CH = HALF // K
NB = K + 1


def kernel(x):
    x = pltpu.with_memory_space_constraint(x, pltpu.MemorySpace.HBM)

    def body(x_ref, out_ref, sendbuf, x_ready, diag_sem, in_sem,
             send_a, recv_a, send_b, recv_b):
        my_x = lax.axis_index("x")
        my_y = lax.axis_index("y")
        peer_x = 1 - my_x
        peer_y = 1 - my_y

        barrier_sem = pltpu.get_barrier_semaphore()
        pl.semaphore_signal(
            barrier_sem,
            inc=1,
            device_id=(my_x, peer_y),
            device_id_type=pl.DeviceIdType.MESH,
        )
        pl.semaphore_signal(
            x_ready,
            inc=1,
            device_id=(peer_x, my_y),
            device_id_type=pl.DeviceIdType.MESH,
        )

        stage_in = pltpu.make_async_copy(
            x_ref.at[pl.ds(my_x * HALF, HALF), pl.ds(peer_y * B, B)],
            sendbuf,
            in_sem,
        )
        stage_in.start()

        diag = pltpu.make_async_copy(
            x_ref.at[:, pl.ds(my_y * B, B)],
            out_ref.at[pl.ds(my_y * B, B), :],
            diag_sem,
        )
        diag.start()

        pl.semaphore_wait(barrier_sem, 1)
        stage_in.wait()

        rdma_a = []
        for c in range(K):
            a = pltpu.make_async_remote_copy(
                src_ref=sendbuf.at[pl.ds(c * CH, CH), :],
                dst_ref=out_ref.at[
                    pl.ds(my_y * B + my_x * HALF + c * CH, CH), :],
                send_sem=send_a.at[c],
                recv_sem=recv_a.at[c],
                device_id=(my_x, peer_y),
                device_id_type=pl.DeviceIdType.MESH,
            )
            a.start()
            rdma_a.append(a)

        pieces = [(c * CH, CH) for c in range(K - 1)]
        pieces += [((K - 1) * CH, CH // 2), ((K - 1) * CH + CH // 2, CH // 2)]
        rdma_b = []
        for i, (off, rows) in enumerate(pieces):
            c = min(off // CH, K - 1)
            if i == 0 or pieces[i - 1][0] // CH != c:
                rdma_a[c].wait_recv()
            if i == 0:
                pl.semaphore_wait(x_ready, 1)
            r = peer_y * B + my_x * HALF + off
            b = pltpu.make_async_remote_copy(
                src_ref=out_ref.at[pl.ds(r, rows), :],
                dst_ref=out_ref.at[pl.ds(r, rows), :],
                send_sem=send_b.at[i],
                recv_sem=recv_b.at[i],
                device_id=(peer_x, my_y),
                device_id_type=pl.DeviceIdType.MESH,
            )
            b.start()
            rdma_b.append(b)

        for b in rdma_b:
            b.wait_recv()
        diag.wait()
        for a in rdma_a:
            a.wait_send()
        for b in rdma_b:
            b.wait_send()

    out = pl.pallas_call(
        body,
        out_shape=jax.ShapeDtypeStruct((2 * B, B), jnp.float32),
        in_specs=[pl.BlockSpec(memory_space=pl.ANY)],
        out_specs=pl.BlockSpec(memory_space=pl.ANY),
        scratch_shapes=[
            pltpu.VMEM((HALF, B), jnp.float32),
            pltpu.SemaphoreType.REGULAR,
            pltpu.SemaphoreType.DMA,
            pltpu.SemaphoreType.DMA,
            pltpu.SemaphoreType.DMA((K,)),
            pltpu.SemaphoreType.DMA((K,)),
            pltpu.SemaphoreType.DMA((NB,)),
            pltpu.SemaphoreType.DMA((NB,)),
        ],
        compiler_params=pltpu.CompilerParams(collective_id=0),
    )(x)
    return out
